# baseline (speedup 1.0000x reference)
import functools

import jax
import jax.numpy as jnp
from jax import lax
from jax.experimental import pallas as pl
from jax.experimental.pallas import tpu as pltpu
from jax.experimental.pallas import tpu_sc as plsc


def _embedding_gather(idx_flat, weight, *, num_cores, num_subcores, nchunk):
    n = idx_flat.shape[0]
    v, d = weight.shape
    nw = num_cores * num_subcores
    b_per_w = n // nw
    chunk = b_per_w // nchunk

    mesh = plsc.VectorSubcoreMesh(core_axis_name="c", subcore_axis_name="s")

    @functools.partial(
        pl.kernel,
        mesh=mesh,
        out_type=jax.ShapeDtypeStruct((n, d), jnp.float32),
        scratch_types=[
            pltpu.VMEM((b_per_w,), jnp.int32),
            pltpu.VMEM((chunk, d), jnp.float32),
            pltpu.VMEM((chunk, d), jnp.float32),
            pltpu.SemaphoreType.DMA,
            pltpu.SemaphoreType.DMA,
        ],
        compiler_params=pltpu.CompilerParams(use_tc_tiling_on_sc=False),
    )
    def k(idx_hbm, table_hbm, out_hbm, idx_v, rows0, rows1, sem0, sem1):
        wid = lax.axis_index("s") * num_cores + lax.axis_index("c")
        base = wid * b_per_w
        pltpu.sync_copy(idx_hbm.at[pl.ds(base, b_per_w)], idx_v)

        bufs = (rows0, rows1)
        sems = (sem0, sem1)

        def gather(c):
            return pltpu.async_copy(
                table_hbm.at[idx_v.at[pl.ds(c * chunk, chunk)]],
                bufs[c % 2],
                sems[c % 2],
            )

        pending = gather(0)
        for c in range(nchunk):
            nxt = gather(c + 1) if c + 1 < nchunk else None
            pending.wait()
            pltpu.sync_copy(bufs[c % 2], out_hbm.at[pl.ds(base + c * chunk, chunk)])
            pending = nxt

    return k(idx_flat, weight)


def kernel(x, weight):
    b, f = x.shape
    n = b * f
    idx_flat = x.reshape(n).astype(jnp.int32)
    out = _embedding_gather(
        idx_flat, weight, num_cores=2, num_subcores=16, nchunk=8
    )
    return out.reshape(b, f, weight.shape[1])


# speedup vs baseline: 1.6037x; 1.6037x over previous
import functools

import jax
import jax.numpy as jnp
from jax import lax
from jax.experimental import pallas as pl
from jax.experimental.pallas import tpu as pltpu
from jax.experimental.pallas import tpu_sc as plsc

_LANES = 16


def _embedding_planes(xt, weight, *, num_cores, num_subcores):
    f, b = xt.shape
    v, d = weight.shape
    nw = num_cores * num_subcores
    nb = b // nw

    mesh = plsc.VectorSubcoreMesh(core_axis_name="c", subcore_axis_name="s")

    @functools.partial(
        pl.kernel,
        mesh=mesh,
        out_type=jax.ShapeDtypeStruct((f, d, b), jnp.float32),
        scratch_types=[
            pltpu.VMEM((f, nb), jnp.int32),
            pltpu.VMEM((nb, d), jnp.float32),
            pltpu.VMEM((nb, d), jnp.float32),
            pltpu.VMEM((d, nb), jnp.float32),
            pltpu.VMEM((d, nb), jnp.float32),
            pltpu.SemaphoreType.DMA,
            pltpu.SemaphoreType.DMA,
            pltpu.SemaphoreType.DMA,
            pltpu.SemaphoreType.DMA,
        ],
        compiler_params=pltpu.CompilerParams(
            use_tc_tiling_on_sc=False, needs_layout_passes=False
        ),
    )
    def k(xt_hbm, table_hbm, out_hbm,
          idx_v, rows0, rows1, tb0, tb1, g0, g1, s0, s1):
        wid = lax.axis_index("s") * num_cores + lax.axis_index("c")
        base = wid * nb
        pltpu.sync_copy(xt_hbm.at[:, pl.ds(base, nb)], idx_v)

        rows = (rows0, rows1)
        tbs = (tb0, tb1)
        gsems = (g0, g1)
        ssems = (s0, s1)

        def gather(fi):
            return pltpu.async_copy(
                table_hbm.at[idx_v.at[fi]], rows[fi % 2], gsems[fi % 2]
            )

        pending = gather(0)
        stores = [None, None]
        for fi in range(f):
            nxt = gather(fi + 1) if fi + 1 < f else None
            pending.wait()
            if stores[fi % 2] is not None:
                stores[fi % 2].wait()
            r = rows[fi % 2]
            t = tbs[fi % 2]

            def transpose_block(g, carry):
                rid = g * _LANES + lax.iota(jnp.int32, _LANES)
                for h in range(d):
                    col = jnp.full((_LANES,), h, jnp.int32)
                    t[h, pl.ds(g * _LANES, _LANES)] = plsc.load_gather(
                        r, [rid, col]
                    )
                return carry

            lax.fori_loop(0, nb // _LANES, transpose_block, 0)
            stores[fi % 2] = pltpu.async_copy(
                t, out_hbm.at[fi, :, pl.ds(base, nb)], ssems[fi % 2]
            )
            pending = nxt
        for st in stores:
            if st is not None:
                st.wait()

    return k(xt, weight)


def kernel(x, weight):
    b, f = x.shape
    xt = x.T.astype(jnp.int32)
    out_planes = _embedding_planes(xt, weight, num_cores=2, num_subcores=16)
    return jnp.transpose(out_planes, (2, 0, 1))
